# baseline (speedup 1.0000x reference)
import functools

import jax
import jax.numpy as jnp
from jax import lax
from jax.experimental import pallas as pl
from jax.experimental.pallas import tpu as pltpu
from jax.experimental.pallas import tpu_sc as plsc

NUM_SC_CORES = 2
NUM_SUBCORES = 16
NW = NUM_SC_CORES * NUM_SUBCORES
BATCH = 16384
EMBED = 64
NROWS = 1000000
RW = 8192
NBLK4 = 32
SPLIT4 = NBLK4 * RW
LASTBLK = (NROWS - 1) // RW
B_PER_W = BATCH // NW
CHUNK = 128
N_CHUNKS = B_PER_W // CHUNK


def _rne16(t):
    u = lax.bitcast_convert_type(t, jnp.uint32)
    return (u + jnp.uint32(0x7FFF) + ((u >> 16) & jnp.uint32(1))) >> 16


def _repack_body(t0, t1, t2, t3, out):
    z01 = (_rne16(t0[...]) | (_rne16(t1[...]) << 16)).T
    z23 = (_rne16(t2[...]) | (_rne16(t3[...]) << 16)).T
    out[...] = lax.bitcast_convert_type(
        jnp.concatenate([z01, z23], axis=1), jnp.int32)


def _tc_repack(tableT):
    def mk(e):
        return pl.BlockSpec(
            (EMBED, RW),
            lambda i, e=e: (0, jnp.minimum(i + e * NBLK4, LASTBLK)))
    return pl.pallas_call(
        _repack_body,
        grid=(NBLK4,),
        in_specs=[mk(0), mk(1), mk(2), mk(3)],
        out_specs=pl.BlockSpec((RW, 128), lambda i: (i, 0)),
        out_shape=jax.ShapeDtypeStruct((SPLIT4, 128), jnp.int32),
    )(tableT, tableT, tableT, tableT)


def _gather_body(tbl_hbm, idx_hbm, out_hbm, idx_v, rows_v, sem):
    wid = lax.axis_index("s") * NUM_SC_CORES + lax.axis_index("c")
    base = wid * B_PER_W
    pltpu.sync_copy(idx_hbm.at[wid], idx_v)
    copies = [
        pltpu.async_copy(tbl_hbm.at[idx_v.at[j]],
                         rows_v.at[pl.ds(j * CHUNK, CHUNK)], sem)
        for j in range(N_CHUNKS)
    ]
    for c in copies:
        c.wait()
    pltpu.sync_copy(rows_v, out_hbm.at[pl.ds(base, B_PER_W)])


def _sc_gather(packed, idx):
    mesh = plsc.VectorSubcoreMesh(core_axis_name="c", subcore_axis_name="s")
    k = pl.kernel(
        _gather_body,
        out_type=jax.ShapeDtypeStruct((BATCH, 128), jnp.int32),
        mesh=mesh,
        scratch_types=[
            pltpu.VMEM((N_CHUNKS, CHUNK), jnp.int32),
            pltpu.VMEM((B_PER_W, 128), jnp.int32),
            pltpu.SemaphoreType.DMA,
        ],
    )
    return k(packed, idx)


BLK = 2048


def _select4(g_i32, e):
    gh = jnp.where(e >= 2, g_i32[:, EMBED:], g_i32[:, :EMBED])
    bits = jnp.where((e & 1) > 0,
                     gh & jnp.int32(-65536), gh << 16)
    return lax.bitcast_convert_type(bits, jnp.float32)


def _mlp_body(gu, gi, eu, ei, w1u, w1i, b1, w2, b2, w3, b3, w4, b4, out):
    xu = _select4(gu[...], eu[...])
    xi = _select4(gi[...], ei[...])
    h = (jnp.dot(xu, w1u[...], preferred_element_type=jnp.float32)
         + jnp.dot(xi, w1i[...], preferred_element_type=jnp.float32))
    h = jnp.maximum(h + b1[...], 0.0)
    h = jnp.maximum(
        jnp.dot(h, w2[...], preferred_element_type=jnp.float32) + b2[...], 0.0)
    h = jnp.maximum(
        jnp.dot(h, w3[...], preferred_element_type=jnp.float32) + b3[...], 0.0)
    out[...] = jnp.sum(h * w4[...], axis=1) + b4[0, 0]


def _tc_mlp(gu, gi, eu, ei, w1u, w1i, b1, w2, b2, w3, b3, w4, b4):
    grid = (BATCH // BLK,)
    full = lambda i: (0, 0)
    col = lambda i: (i, 0)
    return pl.pallas_call(
        _mlp_body,
        grid=grid,
        in_specs=[
            pl.BlockSpec((BLK, 128), col),
            pl.BlockSpec((BLK, 128), col),
            pl.BlockSpec((BLK, 1), col),
            pl.BlockSpec((BLK, 1), col),
            pl.BlockSpec((EMBED, 128), full),
            pl.BlockSpec((EMBED, 128), full),
            pl.BlockSpec((1, 128), full),
            pl.BlockSpec((128, 64), full),
            pl.BlockSpec((1, 64), full),
            pl.BlockSpec((64, 32), full),
            pl.BlockSpec((1, 32), full),
            pl.BlockSpec((1, 32), full),
            pl.BlockSpec((1, 1), full),
        ],
        out_specs=pl.BlockSpec((BLK,), lambda i: (i,)),
        out_shape=jax.ShapeDtypeStruct((BATCH,), jnp.float32),
    )(gu, gi, eu, ei, w1u, w1i, b1, w2, b2, w3, b3, w4, b4)


def kernel(user_indices, item_indices, user_table, item_table,
           W1, b1, W2, b2, W3, b3, W4, b4):
    ui = user_indices.astype(jnp.int32)
    ii = item_indices.astype(jnp.int32)
    eu = ui >> 18
    ei = ii >> 18
    urow = (ui & (SPLIT4 - 1)).reshape(NW, N_CHUNKS, CHUNK)
    irow = (ii & (SPLIT4 - 1)).reshape(NW, N_CHUNKS, CHUNK)

    user_packed = _tc_repack(user_table.T)
    gu = _sc_gather(user_packed, urow)
    item_packed = _tc_repack(item_table.T)
    gi = _sc_gather(item_packed, irow)

    w1t = W1.T
    return _tc_mlp(
        gu, gi, eu.reshape(BATCH, 1), ei.reshape(BATCH, 1),
        w1t[:EMBED], w1t[EMBED:], b1.reshape(1, 128),
        W2.T, b2.reshape(1, 64),
        W3.T, b3.reshape(1, 32),
        W4, b4.reshape(1, 1))


# speedup vs baseline: 1.0173x; 1.0173x over previous
import functools

import jax
import jax.numpy as jnp
from jax import lax
from jax.experimental import pallas as pl
from jax.experimental.pallas import tpu as pltpu
from jax.experimental.pallas import tpu_sc as plsc

NUM_SC_CORES = 2
NUM_SUBCORES = 16
NW = NUM_SC_CORES * NUM_SUBCORES
BATCH = 16384
EMBED = 64
NROWS = 1000000
RW = 8192
NBLK4 = 31
SPLIT4 = NBLK4 * RW
LASTBLK = (NROWS - 1) // RW
B_PER_W = BATCH // NW
CHUNK = 128
N_CHUNKS = B_PER_W // CHUNK


def _rne16(t):
    u = lax.bitcast_convert_type(t, jnp.uint32)
    return (u + jnp.uint32(0x7FFF) + ((u >> 16) & jnp.uint32(1))) >> 16


def _repack_body(t0, t1, t2, t3, out):
    z01 = (_rne16(t0[...]) | (_rne16(t1[...]) << 16)).T
    z23 = (_rne16(t2[...]) | (_rne16(t3[...]) << 16)).T
    out[...] = lax.bitcast_convert_type(
        jnp.concatenate([z01, z23], axis=1), jnp.int32)


def _tc_repack(tableT):
    def mk(e):
        return pl.BlockSpec(
            (EMBED, RW),
            lambda i, e=e: (0, jnp.minimum(i + e * NBLK4, LASTBLK)))
    return pl.pallas_call(
        _repack_body,
        grid=(NBLK4,),
        in_specs=[mk(0), mk(1), mk(2), mk(3)],
        out_specs=pl.BlockSpec((RW, 128), lambda i: (i, 0)),
        out_shape=jax.ShapeDtypeStruct((SPLIT4, 128), jnp.int32),
    )(tableT, tableT, tableT, tableT)


def _gather_body(tbl_hbm, idx_hbm, out_hbm, idx_v, rows_v, sem):
    wid = lax.axis_index("s") * NUM_SC_CORES + lax.axis_index("c")
    base = wid * B_PER_W
    pltpu.sync_copy(idx_hbm.at[wid], idx_v)
    copies = [
        pltpu.async_copy(tbl_hbm.at[idx_v.at[j]],
                         rows_v.at[pl.ds(j * CHUNK, CHUNK)], sem)
        for j in range(N_CHUNKS)
    ]
    for c in copies:
        c.wait()
    pltpu.sync_copy(rows_v, out_hbm.at[pl.ds(base, B_PER_W)])


def _sc_gather(packed, idx):
    mesh = plsc.VectorSubcoreMesh(core_axis_name="c", subcore_axis_name="s")
    k = pl.kernel(
        _gather_body,
        out_type=jax.ShapeDtypeStruct((BATCH, 128), jnp.int32),
        mesh=mesh,
        scratch_types=[
            pltpu.VMEM((N_CHUNKS, CHUNK), jnp.int32),
            pltpu.VMEM((B_PER_W, 128), jnp.int32),
            pltpu.SemaphoreType.DMA,
        ],
    )
    return k(packed, idx)


BLK = 4096


def _select4(g_i32, e):
    gh = jnp.where(e >= 2, g_i32[:, EMBED:], g_i32[:, :EMBED])
    bits = jnp.where((e & 1) > 0,
                     gh & jnp.int32(-65536), gh << 16)
    return lax.bitcast_convert_type(bits, jnp.float32)


def _mlp_body(gu, gi, eu, ei, w1u, w1i, b1, w2, b2, w3, b3, w4, b4, out):
    xu = _select4(gu[...], eu[...])
    xi = _select4(gi[...], ei[...])
    h = (jnp.dot(xu, w1u[...], preferred_element_type=jnp.float32)
         + jnp.dot(xi, w1i[...], preferred_element_type=jnp.float32))
    h = jnp.maximum(h + b1[...], 0.0)
    h = jnp.maximum(
        jnp.dot(h, w2[...], preferred_element_type=jnp.float32) + b2[...], 0.0)
    h = jnp.maximum(
        jnp.dot(h, w3[...], preferred_element_type=jnp.float32) + b3[...], 0.0)
    out[...] = jnp.sum(h * w4[...], axis=1) + b4[0, 0]


def _tc_mlp(gu, gi, eu, ei, w1u, w1i, b1, w2, b2, w3, b3, w4, b4):
    grid = (BATCH // BLK,)
    full = lambda i: (0, 0)
    col = lambda i: (i, 0)
    return pl.pallas_call(
        _mlp_body,
        grid=grid,
        in_specs=[
            pl.BlockSpec((BLK, 128), col),
            pl.BlockSpec((BLK, 128), col),
            pl.BlockSpec((BLK, 1), col),
            pl.BlockSpec((BLK, 1), col),
            pl.BlockSpec((EMBED, 128), full),
            pl.BlockSpec((EMBED, 128), full),
            pl.BlockSpec((1, 128), full),
            pl.BlockSpec((128, 64), full),
            pl.BlockSpec((1, 64), full),
            pl.BlockSpec((64, 32), full),
            pl.BlockSpec((1, 32), full),
            pl.BlockSpec((1, 32), full),
            pl.BlockSpec((1, 1), full),
        ],
        out_specs=pl.BlockSpec((BLK,), lambda i: (i,)),
        out_shape=jax.ShapeDtypeStruct((BATCH,), jnp.float32),
    )(gu, gi, eu, ei, w1u, w1i, b1, w2, b2, w3, b3, w4, b4)


def kernel(user_indices, item_indices, user_table, item_table,
           W1, b1, W2, b2, W3, b3, W4, b4):
    ui = user_indices.astype(jnp.int32)
    ii = item_indices.astype(jnp.int32)
    eu = ui // SPLIT4
    ei = ii // SPLIT4
    urow = (ui - eu * SPLIT4).reshape(NW, N_CHUNKS, CHUNK)
    irow = (ii - ei * SPLIT4).reshape(NW, N_CHUNKS, CHUNK)

    user_packed = _tc_repack(user_table.T)
    gu = _sc_gather(user_packed, urow)
    item_packed = _tc_repack(item_table.T)
    gi = _sc_gather(item_packed, irow)

    w1t = W1.T
    return _tc_mlp(
        gu, gi, eu.reshape(BATCH, 1), ei.reshape(BATCH, 1),
        w1t[:EMBED], w1t[EMBED:], b1.reshape(1, 128),
        W2.T, b2.reshape(1, 64),
        W3.T, b3.reshape(1, 32),
        W4, b4.reshape(1, 1))
